# baseline (speedup 1.0000x reference)
import functools

import jax
import jax.numpy as jnp
from jax import lax
from jax.experimental import pallas as pl
from jax.experimental.pallas import tpu as pltpu
from jax.experimental.pallas import tpu_sc as plsc

DEPTH = 2
DIM_EMB = 128
N_MODULES = 8
EXAMPLES_PER_STEP = 8


def _pool_kernel(x_ref, wctl_ref, bctl_ref, ctl_ref, rows_ref):
    eps = x_ref.shape[0]
    hw = x_ref.shape[2]
    xms = [jnp.sum(x_ref[i], axis=1, keepdims=True) * (1.0 / hw)
           for i in range(eps)]
    xm = jnp.concatenate(xms, axis=1)
    ctl_all = jnp.dot(wctl_ref[...], xm,
                      preferred_element_type=jnp.float32) + bctl_ref[...]
    for t in range(DEPTH):
        ctl_t = ctl_all[t * DIM_EMB:(t + 1) * DIM_EMB, :]
        rows_ref[:, t, :] = ctl_t.T
        for i in range(eps):
            ctl_ref[i, :, t:t + 1] = ctl_t[:, i:i + 1]


_GATHER_1D = lax.GatherDimensionNumbers(
    offset_dims=(), collapsed_slice_dims=(0,), start_index_map=(0,))


def _perm16(v, idx):
    return lax.gather(v, idx[:, None], _GATHER_1D, slice_sizes=(1,),
                      mode=lax.GatherScatterMode.PROMISE_IN_BOUNDS)


def _hsum16(v):
    iota = lax.iota(jnp.int32, 16)
    for sh in (1, 2, 4, 8):
        v = v + _perm16(v, (iota + sh) & 15)
    return v


def _sc_route(rows_hbm, embr_hbm, idx_hbm, row_v, embr_v, idx_v):
    c = lax.axis_index("c")
    s = lax.axis_index("s")
    w = s * 2 + c
    pltpu.sync_copy(rows_hbm.at[w], row_v)
    pltpu.sync_copy(embr_hbm, embr_v)
    best = jnp.full((16,), 3.4e38, jnp.float32)
    bidx = jnp.zeros((16,), jnp.int32)
    for k in range(N_MODULES):
        acc = jnp.zeros((16,), jnp.float32)
        for i in range(DIM_EMB // 16):
            d = row_v[pl.ds(16 * i, 16)] - embr_v[k, pl.ds(16 * i, 16)]
            acc = acc + d * d
        dist = _hsum16(acc)
        pred = dist < best
        best = jnp.where(pred, dist, best)
        bidx = jnp.where(pred, jnp.full((16,), k, jnp.int32), bidx)
    idx_v[...] = bidx
    pltpu.sync_copy(idx_v, idx_hbm.at[w])


def _apply_kernel(idx_ref, x_ref, embc_ref, wcomp_ref, bcomp_ref,
                  y_ref, ctln_ref):
    eps = x_ref.shape[0]
    step = pl.program_id(0)
    composed = []
    for i in range(eps):
        idx0 = idx_ref[(step * eps + i) * DEPTH + 0]
        idx1 = idx_ref[(step * eps + i) * DEPTH + 1]
        ctln_ref[i, :, 0:1] = embc_ref[idx0]
        ctln_ref[i, :, 1:2] = embc_ref[idx1]
        w1 = wcomp_ref[idx0]
        w2 = wcomp_ref[idx1]
        b1 = bcomp_ref[idx0]
        b2 = bcomp_ref[idx1]
        w_eff = jnp.dot(w2, w1, preferred_element_type=jnp.float32)
        b_eff = jnp.dot(w2, b1, preferred_element_type=jnp.float32) + b2
        composed.append((w_eff.astype(jnp.bfloat16), b_eff))
    for i in range(eps):
        w_eff, b_eff = composed[i]
        y = jnp.dot(w_eff, x_ref[i].astype(jnp.bfloat16),
                    preferred_element_type=jnp.float32)
        y_ref[i] = (y + b_eff).astype(jnp.bfloat16)


def kernel(x, W_ctl, b_ctl, emb, W_comp, b_comp):
    Bn, C, H, W = x.shape
    HW = H * W
    x2 = x.reshape(Bn, C, HW)
    W_ctl_dm = (W_ctl.reshape(DIM_EMB, DEPTH, C)
                .transpose(1, 0, 2).reshape(DEPTH * DIM_EMB, C))
    b_ctl_dm = b_ctl.reshape(DIM_EMB, DEPTH).T.reshape(DEPTH * DIM_EMB, 1)
    emb_rows = emb.T
    emb_cols = emb.T.reshape(N_MODULES, DIM_EMB, 1)
    b_comp_c = b_comp.reshape(N_MODULES, C, 1)
    eps = EXAMPLES_PER_STEP

    ctl, rows = pl.pallas_call(
        _pool_kernel,
        grid=(Bn // eps,),
        in_specs=[
            pl.BlockSpec((eps, C, HW), lambda e: (e, 0, 0)),
            pl.BlockSpec((DEPTH * DIM_EMB, C), lambda e: (0, 0)),
            pl.BlockSpec((DEPTH * DIM_EMB, 1), lambda e: (0, 0)),
        ],
        out_specs=[
            pl.BlockSpec((eps, DIM_EMB, DEPTH), lambda e: (e, 0, 0)),
            pl.BlockSpec((eps, DEPTH, DIM_EMB), lambda e: (e, 0, 0)),
        ],
        out_shape=[
            jax.ShapeDtypeStruct((Bn, DIM_EMB, DEPTH), jnp.float32),
            jax.ShapeDtypeStruct((Bn, DEPTH, DIM_EMB), jnp.float32),
        ],
        compiler_params=pltpu.CompilerParams(
            dimension_semantics=("arbitrary",),
        ),
    )(x2, W_ctl_dm, b_ctl_dm)

    mesh = plsc.VectorSubcoreMesh(core_axis_name="c", subcore_axis_name="s")
    idx16 = pl.kernel(
        _sc_route,
        mesh=mesh,
        out_type=jax.ShapeDtypeStruct((Bn * DEPTH, 16), jnp.int32),
        scratch_types=[
            pltpu.VMEM((DIM_EMB,), jnp.float32),
            pltpu.VMEM((N_MODULES, DIM_EMB), jnp.float32),
            pltpu.VMEM((16,), jnp.int32),
        ],
    )(rows.reshape(Bn * DEPTH, DIM_EMB), emb_rows)
    idx_flat = idx16[:, 0]

    y, ctln = pl.pallas_call(
        _apply_kernel,
        grid=(Bn // eps,),
        in_specs=[
            pl.BlockSpec(memory_space=pltpu.SMEM),
            pl.BlockSpec((eps, C, HW), lambda e: (e, 0, 0)),
            pl.BlockSpec((N_MODULES, DIM_EMB, 1), lambda e: (0, 0, 0)),
            pl.BlockSpec((N_MODULES, C, C), lambda e: (0, 0, 0)),
            pl.BlockSpec((N_MODULES, C, 1), lambda e: (0, 0, 0)),
        ],
        out_specs=[
            pl.BlockSpec((eps, C, HW), lambda e: (e, 0, 0)),
            pl.BlockSpec((eps, DIM_EMB, DEPTH), lambda e: (e, 0, 0)),
        ],
        out_shape=[
            jax.ShapeDtypeStruct((Bn, C, HW), jnp.bfloat16),
            jax.ShapeDtypeStruct((Bn, DIM_EMB, DEPTH), jnp.float32),
        ],
        compiler_params=pltpu.CompilerParams(
            dimension_semantics=("arbitrary",),
        ),
    )(idx_flat, x2, emb_cols, W_comp, b_comp_c)
    return (y.astype(jnp.float32).reshape(Bn, C, H, W), ctl, ctln)


# speedup vs baseline: 1.3386x; 1.3386x over previous
import jax
import jax.numpy as jnp
from jax import lax
from jax.experimental import pallas as pl
from jax.experimental.pallas import tpu as pltpu

DEPTH = 2
DIM_EMB = 128
N_MODULES = 8


def _argmin8(score):
    k = score.shape[-1]
    min_s = jnp.min(score)
    iota = lax.broadcasted_iota(jnp.int32, score.shape, 1)
    return jnp.min(jnp.where(score == min_s, iota, k))


EXAMPLES_PER_STEP = 8


def _fused_kernel(x_ref, wctl_ref, bctl_ref, emb_ref, embc_ref,
                  wcomp_ref, bcomp_ref, y_ref, ctl_ref, ctln_ref):
    eps = EXAMPLES_PER_STEP
    hw = x_ref.shape[2]
    e2 = jnp.sum(emb_ref[...] ** 2, axis=0, keepdims=True)

    xms = [jnp.sum(x_ref[i], axis=1, keepdims=True) * (1.0 / hw)
           for i in range(eps)]
    xm = jnp.concatenate(xms, axis=1)

    ctl_all = jnp.dot(wctl_ref[...], xm,
                      preferred_element_type=jnp.float32) + bctl_ref[...]

    idxs = []
    for t in range(DEPTH):
        ctl_t = ctl_all[t * DIM_EMB:(t + 1) * DIM_EMB, :]
        dots = lax.dot_general(ctl_t, emb_ref[...], (((0,), (0,)), ((), ())),
                               preferred_element_type=jnp.float32)
        score = e2 - 2.0 * dots
        min_s = jnp.min(score, axis=1, keepdims=True)
        iota = lax.broadcasted_iota(jnp.int32, score.shape, 1)
        idxs.append(jnp.min(jnp.where(score == min_s, iota, score.shape[1]),
                            axis=1))

    for i in range(eps):
        ctl_ref[i, :, 0:1] = ctl_all[0 * DIM_EMB:1 * DIM_EMB, i:i + 1]
        ctl_ref[i, :, 1:2] = ctl_all[1 * DIM_EMB:2 * DIM_EMB, i:i + 1]

    composed = []
    for i in range(eps):
        idx0 = idxs[0][i]
        idx1 = idxs[1][i]
        ctln_ref[i, :, 0:1] = embc_ref[idx0]
        ctln_ref[i, :, 1:2] = embc_ref[idx1]
        w1 = wcomp_ref[idx0]
        w2 = wcomp_ref[idx1]
        b1 = bcomp_ref[idx0]
        b2 = bcomp_ref[idx1]
        w_eff = jnp.dot(w2, w1, preferred_element_type=jnp.float32)
        b_eff = jnp.dot(w2, b1, preferred_element_type=jnp.float32) + b2
        composed.append((w_eff.astype(jnp.bfloat16), b_eff))

    for i in range(eps):
        w_eff, b_eff = composed[i]
        y = jnp.dot(w_eff, x_ref[i].astype(jnp.bfloat16),
                    preferred_element_type=jnp.float32)
        y_ref[i] = (y + b_eff).astype(jnp.bfloat16)


def kernel(x, W_ctl, b_ctl, emb, W_comp, b_comp):
    Bn, C, H, W = x.shape
    HW = H * W
    x2 = x.reshape(Bn, C, HW)
    W_ctl_dm = (W_ctl.reshape(DIM_EMB, DEPTH, C)
                .transpose(1, 0, 2).reshape(DEPTH * DIM_EMB, C))
    b_ctl_dm = b_ctl.reshape(DIM_EMB, DEPTH).T.reshape(DEPTH * DIM_EMB, 1)
    emb_cols = emb.T.reshape(N_MODULES, DIM_EMB, 1)
    b_comp_c = b_comp.reshape(N_MODULES, C, 1)

    eps = EXAMPLES_PER_STEP
    y, ctl, ctln = pl.pallas_call(
        _fused_kernel,
        grid=(Bn // eps,),
        in_specs=[
            pl.BlockSpec((eps, C, HW), lambda e: (e, 0, 0)),
            pl.BlockSpec((DEPTH * DIM_EMB, C), lambda e: (0, 0)),
            pl.BlockSpec((DEPTH * DIM_EMB, 1), lambda e: (0, 0)),
            pl.BlockSpec((DIM_EMB, N_MODULES), lambda e: (0, 0)),
            pl.BlockSpec((N_MODULES, DIM_EMB, 1), lambda e: (0, 0, 0)),
            pl.BlockSpec((N_MODULES, C, C), lambda e: (0, 0, 0)),
            pl.BlockSpec((N_MODULES, C, 1), lambda e: (0, 0, 0)),
        ],
        out_specs=[
            pl.BlockSpec((eps, C, HW), lambda e: (e, 0, 0)),
            pl.BlockSpec((eps, DIM_EMB, DEPTH), lambda e: (e, 0, 0)),
            pl.BlockSpec((eps, DIM_EMB, DEPTH), lambda e: (e, 0, 0)),
        ],
        out_shape=[
            jax.ShapeDtypeStruct((Bn, C, HW), jnp.bfloat16),
            jax.ShapeDtypeStruct((Bn, DIM_EMB, DEPTH), jnp.float32),
            jax.ShapeDtypeStruct((Bn, DIM_EMB, DEPTH), jnp.float32),
        ],
        compiler_params=pltpu.CompilerParams(
            dimension_semantics=("arbitrary",),
        ),
    )(x2, W_ctl_dm, b_ctl_dm, emb, emb_cols, W_comp, b_comp_c)
    return (y.astype(jnp.float32).reshape(Bn, C, H, W), ctl, ctln)
